# baseline (speedup 1.0000x reference)
import functools

import jax
import jax.numpy as jnp
import numpy as np
from jax import lax
from jax.experimental import pallas as pl
from jax.experimental.pallas import tpu as pltpu
from jax.experimental.pallas import tpu_sc as plsc

N_NODES = 10000
D_IN = 128
HID = 64
N_EDGES = 320000

NC = 2
NS = 16
L = 16
NW = NC * NS
E_PER_W = N_EDGES // NW
CHUNK = 80
N_CHUNKS = E_PER_W // CHUNK
GROUPS = CHUNK // L
STAGE_ROWS = N_NODES // NS


def _proj_body(e_ref, wa_ref, wb_ref, b1_ref, pa_ref, pb_ref):
    e = e_ref[...]
    pa_ref[...] = (
        jnp.dot(e, wa_ref[...], preferred_element_type=jnp.float32) + b1_ref[...]
    ).astype(jnp.bfloat16)
    pb_ref[...] = jnp.dot(
        e, wb_ref[...], preferred_element_type=jnp.float32
    ).astype(jnp.bfloat16)


def _project(node_embeddings, W1, b1):
    return pl.pallas_call(
        _proj_body,
        out_shape=[
            jax.ShapeDtypeStruct((N_NODES, HID), jnp.bfloat16),
            jax.ShapeDtypeStruct((N_NODES, HID), jnp.bfloat16),
        ],
    )(node_embeddings, W1[:D_IN], W1[D_IN:], b1.reshape(1, HID))


def _edge_body(pa_hbm, pb_hbm, ei_hbm, wb_hbm, out_hbm,
               pa_s, pb_s, isrc_v, idst_v, ha0, ha1, w2_v, out_v,
               sem_a0, sem_b0, sem_a1, sem_b1):
    sid = lax.axis_index("s")
    wid = sid * NC + lax.axis_index("c")

    pltpu.sync_copy(pa_hbm.at[pl.ds(sid * STAGE_ROWS, STAGE_ROWS)],
                    pa_s.at[pl.ds(sid * STAGE_ROWS, STAGE_ROWS)])
    pltpu.sync_copy(pb_hbm.at[pl.ds(sid * STAGE_ROWS, STAGE_ROWS)],
                    pb_s.at[pl.ds(sid * STAGE_ROWS, STAGE_ROWS)])
    pltpu.sync_copy(ei_hbm.at[0, pl.ds(wid * E_PER_W, E_PER_W)], isrc_v)
    pltpu.sync_copy(ei_hbm.at[1, pl.ds(wid * E_PER_W, E_PER_W)], idst_v)
    pltpu.sync_copy(wb_hbm, w2_v)
    plsc.subcore_barrier()

    w2c = [w2_v[pl.ds(k * L, L)] for k in range(HID // L)]
    b2 = w2_v[pl.ds(HID, L)]
    lane = lax.iota(jnp.int32, L)
    perms = [lane ^ (1 << p) for p in range(4)]
    masks = [(lane & (1 << p)) != 0 for p in range(4)]

    def start_a(c, h, sa):
        pltpu.async_copy(pa_s.at[isrc_v.at[pl.ds(c * CHUNK, CHUNK)]], h, sa)

    def start_b(c, h, sb):
        pltpu.async_copy(pb_s.at[idst_v.at[pl.ds(c * CHUNK, CHUNK)]], h, sb,
                         add=True)

    def wait_a(c, h, sa):
        pltpu.make_async_copy(
            pa_s.at[isrc_v.at[pl.ds(c * CHUNK, CHUNK)]], h, sa).wait()

    def wait_b(c, h, sb):
        pltpu.make_async_copy(
            pb_s.at[idst_v.at[pl.ds(c * CHUNK, CHUNK)]], h, sb).wait()

    def merge(x, y, p):
        m = masks[p]
        xa = jnp.where(m, y, x)
        yb = jnp.where(m, x, y)
        return xa + yb.at[perms[p]].get(mode="promise_in_bounds")

    def compute(c, h_v):
        def group_body(g, carry2):
            base = g * L
            ts = []
            for i in range(L):
                e = base + i
                t = jnp.zeros((L,), jnp.float32)
                for k in range(HID // (2 * L)):
                    h_bf = h_v[e, pl.ds(k * 2 * L, 2 * L)]
                    h_ev, h_od = plsc.unpack(h_bf, format=plsc.PackFormat.INTERLEAVED)
                    t = t + jnp.maximum(h_ev, 0.0) * w2c[2 * k]
                    t = t + jnp.maximum(h_od, 0.0) * w2c[2 * k + 1]
                ts.append(t)
            for p in range(4):
                ts = [merge(ts[i], ts[i + 1], p) for i in range(0, len(ts), 2)]
            out_v[pl.ds(c * CHUNK + base, L)] = ts[0] + b2
            return carry2

        lax.fori_loop(0, GROUPS, group_body, 0)

    start_a(0, ha0, sem_a0)

    def pair_body(c2, carry):
        c = c2 * 2
        wait_a(c, ha0, sem_a0)
        start_b(c, ha0, sem_b0)
        start_a(c + 1, ha1, sem_a1)
        wait_b(c, ha0, sem_b0)
        compute(c, ha0)
        wait_a(c + 1, ha1, sem_a1)
        start_b(c + 1, ha1, sem_b1)
        start_a(c + 2, ha0, sem_a0)
        wait_b(c + 1, ha1, sem_b1)
        compute(c + 1, ha1)
        return carry

    lax.fori_loop(0, (N_CHUNKS - 1) // 2, pair_body, 0)
    wait_a(N_CHUNKS - 1, ha0, sem_a0)
    start_b(N_CHUNKS - 1, ha0, sem_b0)
    wait_b(N_CHUNKS - 1, ha0, sem_b0)
    compute(N_CHUNKS - 1, ha0)
    pltpu.sync_copy(out_v, out_hbm.at[pl.ds(wid * E_PER_W, E_PER_W)])


_edge_kernel = functools.partial(
    pl.kernel,
    mesh=plsc.VectorSubcoreMesh(core_axis_name="c", subcore_axis_name="s"),
    out_type=jax.ShapeDtypeStruct((N_EDGES,), jnp.float32),
    compiler_params=pltpu.CompilerParams(
        use_tc_tiling_on_sc=False, needs_layout_passes=False
    ),
    scratch_types=[
        pltpu.VMEM_SHARED((N_NODES, HID), jnp.bfloat16),
        pltpu.VMEM_SHARED((N_NODES, HID), jnp.bfloat16),
        pltpu.VMEM((E_PER_W,), jnp.int32),
        pltpu.VMEM((E_PER_W,), jnp.int32),
        pltpu.VMEM((CHUNK, HID), jnp.bfloat16),
        pltpu.VMEM((CHUNK, HID), jnp.bfloat16),
        pltpu.VMEM((HID + L,), jnp.float32),
        pltpu.VMEM((E_PER_W,), jnp.float32),
        pltpu.SemaphoreType.DMA,
        pltpu.SemaphoreType.DMA,
        pltpu.SemaphoreType.DMA,
        pltpu.SemaphoreType.DMA,
    ],
)(_edge_body)


_W2_ORDER = np.arange(HID).reshape(HID // (2 * L), L, 2)
_W2_ORDER = np.concatenate(
    [np.concatenate([blk[:, 0], blk[:, 1]]) for blk in _W2_ORDER]
)


def kernel(node_embeddings, edge_index, W1, b1, W2, b2):
    pa, pb = _project(node_embeddings, W1, b1)
    ei = edge_index.astype(jnp.int32)
    wb = jnp.concatenate(
        [W2.reshape(HID)[_W2_ORDER], jnp.full((L,), b2[0], jnp.float32)]
    )
    out = _edge_kernel(pa, pb, ei, wb)
    return out.reshape(N_EDGES, 1)


# speedup vs baseline: 1.1425x; 1.1425x over previous
import functools

import jax
import jax.numpy as jnp
import numpy as np
from jax import lax
from jax.experimental import pallas as pl
from jax.experimental.pallas import tpu as pltpu
from jax.experimental.pallas import tpu_sc as plsc

N_NODES = 10000
D_IN = 128
HID = 64
N_EDGES = 320000

NC = 2
NS = 16
L = 16
NW = NC * NS
E_PER_W = N_EDGES // NW
CHUNK = 80
N_CHUNKS = E_PER_W // CHUNK
GROUPS = CHUNK // L
STAGE_ROWS = N_NODES // NS


def _proj_body(e_ref, wa_ref, wb_ref, b1_ref, pa_ref, pb_ref):
    e = e_ref[...]
    pa_ref[...] = (
        jnp.dot(e, wa_ref[...], preferred_element_type=jnp.float32) + b1_ref[...]
    ).astype(jnp.bfloat16)
    pb_ref[...] = jnp.dot(
        e, wb_ref[...], preferred_element_type=jnp.float32
    ).astype(jnp.bfloat16)


def _project(node_embeddings, W1, b1):
    return pl.pallas_call(
        _proj_body,
        out_shape=[
            jax.ShapeDtypeStruct((N_NODES, HID), jnp.bfloat16),
            jax.ShapeDtypeStruct((N_NODES, HID), jnp.bfloat16),
        ],
    )(node_embeddings, W1[:D_IN], W1[D_IN:], b1.reshape(1, HID))


def _edge_body(pa_hbm, pb_hbm, ei_hbm, wb_hbm, out_hbm,
               pa_s, pb_s, isrc_v, idst_v, ha0, ha1, w2_v, out_v,
               sem_a0, sem_b0, sem_a1, sem_b1):
    sid = lax.axis_index("s")
    wid = sid * NC + lax.axis_index("c")

    pltpu.sync_copy(pa_hbm.at[pl.ds(sid * STAGE_ROWS, STAGE_ROWS)],
                    pa_s.at[pl.ds(sid * STAGE_ROWS, STAGE_ROWS)])
    pltpu.sync_copy(pb_hbm.at[pl.ds(sid * STAGE_ROWS, STAGE_ROWS)],
                    pb_s.at[pl.ds(sid * STAGE_ROWS, STAGE_ROWS)])
    pltpu.sync_copy(ei_hbm.at[0, pl.ds(wid * E_PER_W, E_PER_W)], isrc_v)
    pltpu.sync_copy(ei_hbm.at[1, pl.ds(wid * E_PER_W, E_PER_W)], idst_v)
    pltpu.sync_copy(wb_hbm, w2_v)
    plsc.subcore_barrier()

    w2c = [w2_v[pl.ds(k * L, L)] for k in range(HID // L)]
    b2 = w2_v[pl.ds(HID, L)]
    lane = lax.iota(jnp.int32, L)
    perms = [lane ^ (1 << p) for p in range(4)]
    masks = [(lane & (1 << p)) != 0 for p in range(4)]

    def start_a(c, h, sa):
        pltpu.async_copy(pa_s.at[isrc_v.at[pl.ds(c * CHUNK, CHUNK)]], h, sa)

    def start_b(c, h, sb):
        pltpu.async_copy(pb_s.at[idst_v.at[pl.ds(c * CHUNK, CHUNK)]], h, sb,
                         add=True)

    def wait_a(c, h, sa):
        pltpu.make_async_copy(
            pa_s.at[isrc_v.at[pl.ds(c * CHUNK, CHUNK)]], h, sa).wait()

    def wait_b(c, h, sb):
        pltpu.make_async_copy(
            pb_s.at[idst_v.at[pl.ds(c * CHUNK, CHUNK)]], h, sb).wait()

    def merge(x, y, p):
        m = masks[p]
        xa = jnp.where(m, y, x)
        yb = jnp.where(m, x, y)
        return xa + yb.at[perms[p]].get(mode="promise_in_bounds")

    def compute(c, h_v):
        def group_body(g, carry2):
            base = g * L
            ts = []
            for i in range(L):
                e = base + i
                t = jnp.zeros((L,), jnp.float32)
                for k in range(HID // (2 * L)):
                    h_bf = h_v[e, pl.ds(k * 2 * L, 2 * L)]
                    h_ev, h_od = plsc.unpack(h_bf, format=plsc.PackFormat.INTERLEAVED)
                    t = t + jnp.maximum(h_ev, 0.0) * w2c[2 * k]
                    t = t + jnp.maximum(h_od, 0.0) * w2c[2 * k + 1]
                ts.append(t)
            for p in range(4):
                ts = [merge(ts[i], ts[i + 1], p) for i in range(0, len(ts), 2)]
            out_v[pl.ds(c * CHUNK + base, L)] = ts[0] + b2
            return carry2

        lax.fori_loop(0, GROUPS, group_body, 0)

    start_a(0, ha0, sem_a0)

    def pair_body(c2, carry):
        c = c2 * 2
        wait_a(c, ha0, sem_a0)
        start_b(c, ha0, sem_b0)
        start_a(c + 1, ha1, sem_a1)
        wait_b(c, ha0, sem_b0)
        wait_a(c + 1, ha1, sem_a1)
        start_b(c + 1, ha1, sem_b1)
        compute(c, ha0)
        start_a(c + 2, ha0, sem_a0)
        wait_b(c + 1, ha1, sem_b1)
        compute(c + 1, ha1)
        return carry

    lax.fori_loop(0, (N_CHUNKS - 1) // 2, pair_body, 0)
    wait_a(N_CHUNKS - 1, ha0, sem_a0)
    start_b(N_CHUNKS - 1, ha0, sem_b0)
    wait_b(N_CHUNKS - 1, ha0, sem_b0)
    compute(N_CHUNKS - 1, ha0)
    pltpu.sync_copy(out_v, out_hbm.at[pl.ds(wid * E_PER_W, E_PER_W)])


_edge_kernel = functools.partial(
    pl.kernel,
    mesh=plsc.VectorSubcoreMesh(core_axis_name="c", subcore_axis_name="s"),
    out_type=jax.ShapeDtypeStruct((N_EDGES,), jnp.float32),
    compiler_params=pltpu.CompilerParams(
        use_tc_tiling_on_sc=False, needs_layout_passes=False
    ),
    scratch_types=[
        pltpu.VMEM_SHARED((N_NODES, HID), jnp.bfloat16),
        pltpu.VMEM_SHARED((N_NODES, HID), jnp.bfloat16),
        pltpu.VMEM((E_PER_W,), jnp.int32),
        pltpu.VMEM((E_PER_W,), jnp.int32),
        pltpu.VMEM((CHUNK, HID), jnp.bfloat16),
        pltpu.VMEM((CHUNK, HID), jnp.bfloat16),
        pltpu.VMEM((HID + L,), jnp.float32),
        pltpu.VMEM((E_PER_W,), jnp.float32),
        pltpu.SemaphoreType.DMA,
        pltpu.SemaphoreType.DMA,
        pltpu.SemaphoreType.DMA,
        pltpu.SemaphoreType.DMA,
    ],
)(_edge_body)


_W2_ORDER = np.arange(HID).reshape(HID // (2 * L), L, 2)
_W2_ORDER = np.concatenate(
    [np.concatenate([blk[:, 0], blk[:, 1]]) for blk in _W2_ORDER]
)


def kernel(node_embeddings, edge_index, W1, b1, W2, b2):
    pa, pb = _project(node_embeddings, W1, b1)
    ei = edge_index.astype(jnp.int32)
    wb = jnp.concatenate(
        [W2.reshape(HID)[_W2_ORDER], jnp.full((L,), b2[0], jnp.float32)]
    )
    out = _edge_kernel(pa, pb, ei, wb)
    return out.reshape(N_EDGES, 1)
